# baseline (speedup 1.0000x reference)
import functools

import jax
import jax.numpy as jnp
from jax import lax
from jax.experimental import pallas as pl
from jax.experimental.pallas import tpu as pltpu
from jax.experimental.pallas import tpu_sc as plsc

D_EMBED = 64
SCALE = 8.0
LANES = 16
BTILE = 128
NBUF = 3
NTB = 2


def _iota16():
    return lax.broadcasted_iota(jnp.int32, (LANES,), 0)


@functools.lru_cache(maxsize=None)
def _make_k1(vocab):
    info = plsc.get_sparse_core_info()
    nc, ns = info.num_cores, info.num_subcores
    nw = nc * ns
    nblk_full = vocab // BTILE
    tail = vocab - nblk_full * BTILE
    nb_base = nblk_full // nw
    nb_rem = nblk_full - nb_base * nw
    assert nb_base % 2 == 0

    mesh = plsc.VectorSubcoreMesh(core_axis_name="c", subcore_axis_name="s")

    @functools.partial(
        pl.kernel,
        out_type=jax.ShapeDtypeStruct((vocab // 2, BTILE), jnp.float32),
        mesh=mesh,
        scratch_types=[
            [pltpu.VMEM((D_EMBED, 129), jnp.float32) for _ in range(2)],
            [pltpu.VMEM((D_EMBED, BTILE), jnp.float32) for _ in range(2)],
            pltpu.VMEM((tail // 2, BTILE), jnp.float32),
            [pltpu.SemaphoreType.DMA for _ in range(2)],
            [pltpu.SemaphoreType.DMA for _ in range(2)],
        ],
        compiler_params=pltpu.CompilerParams(
            use_tc_tiling_on_sc=True, needs_layout_passes=False
        ),
    )
    def relayout(tT, tailp, tpc, slabs, rowbufs, tailv, gsems, ssems):
        wid = lax.axis_index("s") * nc + lax.axis_index("c")
        nb = nb_base + jnp.where(wid < nb_rem, 1, 0)

        dv4 = [_iota16() + c * LANES for c in range(4)]
        zero16 = _iota16() * 0

        def bid_of(i):
            return wid + nw * i

        def slab_dst(b):
            return slabs[b].at[:, pl.ds(0, BTILE)]

        def start_slab(i, b):
            v0 = pl.multiple_of(bid_of(i) * BTILE, BTILE)
            pltpu.async_copy(
                tT.at[:, pl.ds(v0, BTILE)], slab_dst(b), gsems[b]
            )

        def wait_slab(i, b):
            v0 = pl.multiple_of(bid_of(i) * BTILE, BTILE)
            pltpu.make_async_copy(
                tT.at[:, pl.ds(v0, BTILE)], slab_dst(b), gsems[b]
            ).wait()

        def store_pair(i, b):
            p0 = bid_of(i) * (BTILE // 2)
            pltpu.async_copy(
                rowbufs[b], tpc.at[pl.ds(p0, 64)], ssems[b]
            )

        def wait_store_pair(i, b):
            p0 = bid_of(i) * (BTILE // 2)
            pltpu.make_async_copy(
                rowbufs[b], tpc.at[pl.ds(p0, 64)], ssems[b]
            ).wait()

        def transpose(slab, rowbuf):
            NP = 2

            def pbody(p2, carry):
                p0 = p2 * NP
                vals = []
                splats = []
                for pp in range(NP):
                    je = zero16 + (2 * (p0 + pp))
                    jo = je + 1
                    splats.append((je, jo))
                    for c in range(8):
                        jv = je if c < 4 else jo
                        vals.append(
                            plsc.load_gather(slab, [dv4[c % 4], jv]) * SCALE
                        )
                for pp in range(NP):
                    for c in range(8):
                        rowbuf[p0 + pp, pl.ds(c * LANES, LANES)] = (
                            vals[pp * 8 + c]
                        )
                return carry

            lax.fori_loop(0, 64 // NP, pbody, 0)

        def block(i, b, prefetch, storewait):
            wait_slab(i, b)
            if prefetch:
                nxt = jnp.minimum(i + 1, nb - 1)
                start_slab(nxt, 1 - b)
            if storewait:
                wait_store_pair(i - 2, b)
            transpose(slabs[b], rowbufs[b])
            store_pair(i, b)

        start_slab(0, 0)
        block(0, 0, True, False)
        block(1, 1, True, False)

        def main(g, carry):
            i0 = 2 + g * 2
            block(i0, 0, True, True)
            block(i0 + 1, 1, True, True)
            return carry

        lax.fori_loop(0, (nb_base - 2) // 2, main, 0)

        @pl.when(wid < nb_rem)
        def _extra():
            block(nb_base, 0, False, True)
            wait_store_pair(nb_base - 1, 1)
            wait_store_pair(nb_base, 0)

        @pl.when(wid >= nb_rem)
        def _drain():
            wait_slab(nb - 1, 0)
            wait_store_pair(nb_base - 2, 0)
            wait_store_pair(nb_base - 1, 1)

        @pl.when(wid == nw - 1)
        def _tail():
            p0 = (nblk_full * BTILE) // 2
            pltpu.sync_copy(tailp, tailv)
            pltpu.sync_copy(tailv, tpc.at[pl.ds(p0, tail // 2)])

    return relayout


@functools.lru_cache(maxsize=None)
def _make_k2(batch, hist, vocab):
    info = plsc.get_sparse_core_info()
    nc, ns = info.num_cores, info.num_subcores
    nw = nc * ns
    assert batch == nw * BTILE
    n_chunks = hist

    mesh = plsc.VectorSubcoreMesh(core_axis_name="c", subcore_axis_name="s")

    @functools.partial(
        pl.kernel,
        out_type=jax.ShapeDtypeStruct(
            (hist, D_EMBED // 8, nw, 8, BTILE), jnp.float32
        ),
        mesh=mesh,
        scratch_types=[
            pltpu.VMEM((hist, BTILE), jnp.int32),
            [pltpu.VMEM((BTILE, D_EMBED), jnp.float32) for _ in range(NBUF)],
            [pltpu.VMEM((D_EMBED // 8, 1, 8, 129), jnp.float32)
             for _ in range(NTB)],
            [pltpu.SemaphoreType.DMA for _ in range(NBUF)],
            [pltpu.SemaphoreType.DMA for _ in range(NTB)],
        ],
        compiler_params=pltpu.CompilerParams(
            use_tc_tiling_on_sc=False, needs_layout_passes=False
        ),
    )
    def gather_t(xw, tab, out5, idxbuf, rows, tbufs, gsems, ssems):
        wid = lax.axis_index("s") * nc + lax.axis_index("c")
        pltpu.sync_copy(xw.at[wid], idxbuf)

        dtc = [
            lax.shift_right_logical(_iota16() + c * LANES, 3)
            for c in range(D_EMBED // LANES)
        ]
        dic = [
            lax.bitwise_and(_iota16() + c * LANES, 7)
            for c in range(D_EMBED // LANES)
        ]
        zero16 = _iota16() * 0

        def start_gather(h, b):
            pltpu.async_copy(tab.at[idxbuf.at[h]], rows[b], gsems[b])

        def wait_gather(h, b):
            pltpu.make_async_copy(
                tab.at[idxbuf.at[h]], rows[b], gsems[b]
            ).wait()

        def out_slice(h):
            return out5.at[h, :, pl.ds(wid, 1)]

        def start_store(h, tb):
            pltpu.async_copy(
                tbufs[tb].at[:, :, :, pl.ds(0, BTILE)], out_slice(h),
                ssems[tb],
            )

        def wait_store(h, tb):
            pltpu.make_async_copy(
                tbufs[tb].at[:, :, :, pl.ds(0, BTILE)], out_slice(h),
                ssems[tb],
            ).wait()

        def transpose(b, tb):
            src = rows[b]
            tbuf = tbufs[tb]
            NB = 4

            def bbody(b4, carry):
                bi0 = b4 * NB
                vals = []
                for bb in range(NB):
                    for c in range(D_EMBED // LANES):
                        vals.append(src[bi0 + bb, pl.ds(c * LANES, LANES)])
                for bb in range(NB):
                    biv = zero16 + (bi0 + bb)
                    for c in range(D_EMBED // LANES):
                        plsc.store_scatter(
                            tbuf,
                            [dtc[c], zero16, dic[c], biv],
                            vals[bb * (D_EMBED // LANES) + c],
                        )
                return carry

            lax.fori_loop(0, BTILE // NB, bbody, 0)

        def iter_body(h, b, tb, pre_h, pre_b, postwait):
            wait_gather(h, b)
            transpose(b, tb)
            if pre_h is not None:
                start_gather(pre_h, pre_b)
            if postwait:
                wait_store(h - NTB, tb)
            start_store(h, tb)

        start_gather(0, 0)
        start_gather(1, 1)

        for h in range(6):
            iter_body(
                h, h % NBUF, h % NTB, h + 2, (h + 2) % NBUF,
                postwait=h >= NTB,
            )

        n_main = (n_chunks - 8) // 6
        assert n_chunks == 8 + 6 * n_main

        def main(g, carry):
            h0 = 6 + g * 6
            for u in range(6):
                iter_body(
                    h0 + u, u % NBUF, u % NTB, h0 + u + 2,
                    (u + 2) % NBUF, True,
                )
            return carry

        lax.fori_loop(0, n_main, main, 0)

        for h in range(n_chunks - 2, n_chunks):
            iter_body(h, h % NBUF, h % NTB, None, None, postwait=True)

        for h in range(n_chunks - NTB, n_chunks):
            wait_store(h, h % NTB)

    return gather_t


def kernel(x, table):
    batch, hist = x.shape
    vocab = table.shape[0]
    info = plsc.get_sparse_core_info()
    nw = info.num_cores * info.num_subcores
    k1 = _make_k1(vocab)
    k2 = _make_k2(batch, hist, vocab)
    ntail = vocab % BTILE
    tailp = (table[vocab - ntail:] * SCALE).reshape(ntail // 2, BTILE)
    tpc = k1(table.T, tailp)
    tab = tpc.reshape(vocab, D_EMBED)
    xw = (
        x.astype(jnp.int32)
        .reshape(nw, BTILE, hist)
        .transpose(0, 2, 1)
    )
    out5 = k2(xw, tab)
    return out5.transpose(2, 4, 0, 1, 3).reshape(batch, hist, D_EMBED)


# speedup vs baseline: 2.3013x; 2.3013x over previous
import functools

import jax
import jax.numpy as jnp
from jax import lax
from jax.experimental import pallas as pl
from jax.experimental.pallas import tpu as pltpu
from jax.experimental.pallas import tpu_sc as plsc

D_EMBED = 64
SCALE = 8.0
LANES = 16
BTILE = 128
NBUF = 3
NTB = 2


def _iota16():
    return lax.broadcasted_iota(jnp.int32, (LANES,), 0)


@functools.lru_cache(maxsize=None)
def _make_k1(vocab):
    info = plsc.get_sparse_core_info()
    nc, ns = info.num_cores, info.num_subcores
    nw = nc * ns
    nblk_full = vocab // BTILE
    tail = vocab - nblk_full * BTILE
    nb_base = nblk_full // nw
    nb_rem = nblk_full - nb_base * nw
    assert nb_base % 2 == 0

    mesh = plsc.VectorSubcoreMesh(core_axis_name="c", subcore_axis_name="s")

    @functools.partial(
        pl.kernel,
        out_type=jax.ShapeDtypeStruct((vocab // 2, BTILE), jnp.float32),
        mesh=mesh,
        scratch_types=[
            [pltpu.VMEM((D_EMBED, 129), jnp.float32) for _ in range(2)],
            [pltpu.VMEM((D_EMBED, BTILE), jnp.float32) for _ in range(2)],
            pltpu.VMEM((tail // 2, BTILE), jnp.float32),
            [pltpu.SemaphoreType.DMA for _ in range(2)],
            [pltpu.SemaphoreType.DMA for _ in range(2)],
        ],
        compiler_params=pltpu.CompilerParams(
            use_tc_tiling_on_sc=True, needs_layout_passes=False
        ),
    )
    def relayout(tT, tailp, tpc, slabs, rowbufs, tailv, gsems, ssems):
        wid = lax.axis_index("s") * nc + lax.axis_index("c")
        nb = nb_base + jnp.where(wid < nb_rem, 1, 0)

        dv4 = [_iota16() + c * LANES for c in range(4)]
        zero16 = _iota16() * 0

        def bid_of(i):
            return wid + nw * i

        def slab_dst(b):
            return slabs[b].at[:, pl.ds(0, BTILE)]

        def start_slab(i, b):
            v0 = pl.multiple_of(bid_of(i) * BTILE, BTILE)
            pltpu.async_copy(
                tT.at[:, pl.ds(v0, BTILE)], slab_dst(b), gsems[b]
            )

        def wait_slab(i, b):
            v0 = pl.multiple_of(bid_of(i) * BTILE, BTILE)
            pltpu.make_async_copy(
                tT.at[:, pl.ds(v0, BTILE)], slab_dst(b), gsems[b]
            ).wait()

        def store_pair(i, b):
            p0 = bid_of(i) * (BTILE // 2)
            pltpu.async_copy(
                rowbufs[b], tpc.at[pl.ds(p0, 64)], ssems[b]
            )

        def wait_store_pair(i, b):
            p0 = bid_of(i) * (BTILE // 2)
            pltpu.make_async_copy(
                rowbufs[b], tpc.at[pl.ds(p0, 64)], ssems[b]
            ).wait()

        def transpose(slab, rowbuf):
            NJ = 2

            def jbody(g, carry):
                j00 = g * NJ
                vals = []
                keys = []
                for jj in range(NJ):
                    jv = lax.bitwise_and((j00 + jj) + _iota16(), 127)
                    pv = lax.shift_right_logical(jv, 1)
                    cb = lax.shift_left(lax.bitwise_and(jv, 1), 6)
                    keys.append((pv, cb))
                    for c in range(4):
                        vals.append(
                            plsc.load_gather(slab, [dv4[c], jv]) * SCALE
                        )
                for jj in range(NJ):
                    pv, cb = keys[jj]
                    for c in range(4):
                        plsc.store_scatter(
                            rowbuf, [pv, cb + dv4[c]], vals[jj * 4 + c]
                        )
                return carry

            lax.fori_loop(0, BTILE // NJ, jbody, 0)

        def block(i, b, prefetch, storewait):
            wait_slab(i, b)
            if prefetch:
                nxt = jnp.minimum(i + 1, nb - 1)
                start_slab(nxt, 1 - b)
            if storewait:
                wait_store_pair(i - 2, b)
            transpose(slabs[b], rowbufs[b])
            store_pair(i, b)

        start_slab(0, 0)
        block(0, 0, True, False)
        block(1, 1, True, False)

        def main(g, carry):
            i0 = 2 + g * 2
            block(i0, 0, True, True)
            block(i0 + 1, 1, True, True)
            return carry

        lax.fori_loop(0, (nb_base - 2) // 2, main, 0)

        @pl.when(wid < nb_rem)
        def _extra():
            block(nb_base, 0, False, True)
            wait_store_pair(nb_base - 1, 1)
            wait_store_pair(nb_base, 0)

        @pl.when(wid >= nb_rem)
        def _drain():
            wait_slab(nb - 1, 0)
            wait_store_pair(nb_base - 2, 0)
            wait_store_pair(nb_base - 1, 1)

        @pl.when(wid == nw - 1)
        def _tail():
            p0 = (nblk_full * BTILE) // 2
            pltpu.sync_copy(tailp, tailv)
            pltpu.sync_copy(tailv, tpc.at[pl.ds(p0, tail // 2)])

    return relayout


@functools.lru_cache(maxsize=None)
def _make_k2(batch, hist, vocab):
    info = plsc.get_sparse_core_info()
    nc, ns = info.num_cores, info.num_subcores
    nw = nc * ns
    assert batch == nw * BTILE
    n_chunks = hist

    mesh = plsc.VectorSubcoreMesh(core_axis_name="c", subcore_axis_name="s")

    @functools.partial(
        pl.kernel,
        out_type=jax.ShapeDtypeStruct(
            (hist, D_EMBED // 8, nw, 8, BTILE), jnp.float32
        ),
        mesh=mesh,
        scratch_types=[
            pltpu.VMEM((hist, BTILE), jnp.int32),
            [pltpu.VMEM((BTILE, D_EMBED), jnp.float32) for _ in range(NBUF)],
            [pltpu.VMEM((D_EMBED // 8, 1, 8, 129), jnp.float32)
             for _ in range(NTB)],
            [pltpu.SemaphoreType.DMA for _ in range(NBUF)],
            [pltpu.SemaphoreType.DMA for _ in range(NTB)],
        ],
        compiler_params=pltpu.CompilerParams(
            use_tc_tiling_on_sc=False, needs_layout_passes=False
        ),
    )
    def gather_t(xw, tab, out5, idxbuf, rows, tbufs, gsems, ssems):
        wid = lax.axis_index("s") * nc + lax.axis_index("c")
        pltpu.sync_copy(xw.at[wid], idxbuf)

        dtc = [
            lax.shift_right_logical(_iota16() + c * LANES, 3)
            for c in range(D_EMBED // LANES)
        ]
        dic = [
            lax.bitwise_and(_iota16() + c * LANES, 7)
            for c in range(D_EMBED // LANES)
        ]
        zero16 = _iota16() * 0

        def start_gather(h, b):
            pltpu.async_copy(tab.at[idxbuf.at[h]], rows[b], gsems[b])

        def wait_gather(h, b):
            pltpu.make_async_copy(
                tab.at[idxbuf.at[h]], rows[b], gsems[b]
            ).wait()

        def out_slice(h):
            return out5.at[h, :, pl.ds(wid, 1)]

        def start_store(h, tb):
            pltpu.async_copy(
                tbufs[tb].at[:, :, :, pl.ds(0, BTILE)], out_slice(h),
                ssems[tb],
            )

        def wait_store(h, tb):
            pltpu.make_async_copy(
                tbufs[tb].at[:, :, :, pl.ds(0, BTILE)], out_slice(h),
                ssems[tb],
            ).wait()

        def transpose(b, tb):
            src = rows[b]
            tbuf = tbufs[tb]
            NB = 4

            def bbody(b4, carry):
                bi0 = b4 * NB
                vals = []
                for bb in range(NB):
                    for c in range(D_EMBED // LANES):
                        vals.append(src[bi0 + bb, pl.ds(c * LANES, LANES)])
                for bb in range(NB):
                    biv = zero16 + (bi0 + bb)
                    for c in range(D_EMBED // LANES):
                        plsc.store_scatter(
                            tbuf,
                            [dtc[c], zero16, dic[c], biv],
                            vals[bb * (D_EMBED // LANES) + c],
                        )
                return carry

            lax.fori_loop(0, BTILE // NB, bbody, 0)

        def iter_body(h, b, tb, pre_h, pre_b, postwait):
            wait_gather(h, b)
            transpose(b, tb)
            if pre_h is not None:
                start_gather(pre_h, pre_b)
            if postwait:
                wait_store(h - NTB, tb)
            start_store(h, tb)

        start_gather(0, 0)
        start_gather(1, 1)

        for h in range(6):
            iter_body(
                h, h % NBUF, h % NTB, h + 2, (h + 2) % NBUF,
                postwait=h >= NTB,
            )

        n_main = (n_chunks - 8) // 6
        assert n_chunks == 8 + 6 * n_main

        def main(g, carry):
            h0 = 6 + g * 6
            for u in range(6):
                iter_body(
                    h0 + u, u % NBUF, u % NTB, h0 + u + 2,
                    (u + 2) % NBUF, True,
                )
            return carry

        lax.fori_loop(0, n_main, main, 0)

        for h in range(n_chunks - 2, n_chunks):
            iter_body(h, h % NBUF, h % NTB, None, None, postwait=True)

        for h in range(n_chunks - NTB, n_chunks):
            wait_store(h, h % NTB)

    return gather_t


def kernel(x, table):
    batch, hist = x.shape
    vocab = table.shape[0]
    info = plsc.get_sparse_core_info()
    nw = info.num_cores * info.num_subcores
    k1 = _make_k1(vocab)
    k2 = _make_k2(batch, hist, vocab)
    ntail = vocab % BTILE
    tailp = (table[vocab - ntail:] * SCALE).reshape(ntail // 2, BTILE)
    tpc = k1(table.T, tailp)
    tab = tpc.reshape(vocab, D_EMBED)
    xw = (
        x.astype(jnp.int32)
        .reshape(nw, BTILE, hist)
        .transpose(0, 2, 1)
    )
    out5 = k2(xw, tab)
    return out5.transpose(2, 4, 0, 1, 3).reshape(batch, hist, D_EMBED)
